# baseline (speedup 1.0000x reference)
import functools

import jax
import jax.numpy as jnp
from jax import lax
from jax.experimental import pallas as pl
from jax.experimental.pallas import tpu as pltpu
from jax.experimental.pallas import tpu_sc as plsc

N_NODES = 100000
D = 128
B = 16384
F1 = 10
F2 = 5
NV = D // 16

PER_W = B // 32
TCH = 128
RT = PER_W // TCH
E1 = 8
R1 = PER_W // E1
E2 = 16
R2 = PER_W // E2


def _reduce_rows(rows_v, out_v, n_elems, fanout):
    def elem_body(e, carry):
        base = e * fanout
        accs = [rows_v[base, pl.ds(v * 16, 16)] for v in range(NV)]
        for j in range(1, fanout):
            for v in range(NV):
                accs[v] = accs[v] + rows_v[base + j, pl.ds(v * 16, 16)]
        for v in range(NV):
            out_v[e, pl.ds(v * 16, 16)] = accs[v]
        return carry

    lax.fori_loop(0, n_elems, elem_body, 0)


_MESH = plsc.VectorSubcoreMesh(core_axis_name="c", subcore_axis_name="s")
_INFO = plsc.get_sparse_core_info()


def _worker_id():
    return lax.axis_index("s") * _INFO.num_cores + lax.axis_index("c")


def _sc_gather_targets(tids, feats):
    @functools.partial(
        pl.kernel,
        out_type=[jax.ShapeDtypeStruct((B, D), jnp.float32)],
        mesh=_MESH,
        scratch_types=[
            pltpu.VMEM((PER_W,), jnp.int32),
            pltpu.VMEM((TCH, D), jnp.float32),
            pltpu.VMEM((TCH, D), jnp.float32),
            pltpu.SemaphoreType.DMA,
            pltpu.SemaphoreType.DMA,
            pltpu.SemaphoreType.DMA,
            pltpu.SemaphoreType.DMA,
        ],
    )
    def sc_kernel(tids_hbm, feats_hbm, ht_hbm, idxT, rowsT0, rowsT1,
                  sg0, sg1, so0, so1):
        wid = _worker_id()
        wbase = wid * PER_W
        sg = (sg0, sg1)
        so = (so0, so1)
        rowsT = (rowsT0, rowsT1)
        pltpu.sync_copy(tids_hbm.at[pl.ds(wbase, PER_W)], idxT)
        for half in range(2):
            pltpu.async_copy(
                feats_hbm.at[idxT.at[pl.ds(half * TCH, TCH)]],
                rowsT[half], sg[half],
            )

        def t_body(rr, carry):
            for half in range(2):
                r = 2 * rr + half
                buf = rowsT[half]
                pltpu.make_async_copy(
                    feats_hbm.at[idxT.at[pl.ds(r * TCH, TCH)]], buf, sg[half]
                ).wait()
                base = wbase + r * TCH
                pltpu.async_copy(buf, ht_hbm.at[pl.ds(base, TCH)], so[half])

                @pl.when(r + 2 < RT)
                def _():
                    pltpu.make_async_copy(
                        buf, ht_hbm.at[pl.ds(base, TCH)], so[half]
                    ).wait()
                    pltpu.async_copy(
                        feats_hbm.at[idxT.at[pl.ds((r + 2) * TCH, TCH)]],
                        buf, sg[half],
                    )

            return carry

        lax.fori_loop(0, RT // 2, t_body, 0)
        for half in range(2):
            r = RT - 2 + half
            pltpu.make_async_copy(
                rowsT[half], ht_hbm.at[pl.ds(wbase + r * TCH, TCH)], so[half]
            ).wait()

    return sc_kernel(tids, feats)[0]


def _sc_gather_neighbors(feats, n1_flat, n2_flat):
    @functools.partial(
        pl.kernel,
        out_type=[
            jax.ShapeDtypeStruct((B, D), jnp.float32),
            jax.ShapeDtypeStruct((B, D), jnp.float32),
        ],
        mesh=_MESH,
        scratch_types=[
            pltpu.VMEM((PER_W * F1,), jnp.int32),
            pltpu.VMEM((E1 * F1, D), jnp.float32),
            pltpu.VMEM((E1 * F1, D), jnp.float32),
            pltpu.VMEM((E1, D), jnp.float32),
            pltpu.VMEM((E1, D), jnp.float32),
            pltpu.VMEM((PER_W * F2,), jnp.int32),
            pltpu.VMEM((E2 * F2, D), jnp.float32),
            pltpu.VMEM((E2 * F2, D), jnp.float32),
            pltpu.VMEM((E2, D), jnp.float32),
            pltpu.VMEM((E2, D), jnp.float32),
            pltpu.SemaphoreType.DMA,
            pltpu.SemaphoreType.DMA,
            pltpu.SemaphoreType.DMA,
            pltpu.SemaphoreType.DMA,
        ],
    )
    def sc_kernel(
        feats_hbm, n1_hbm, n2_hbm,
        s1_hbm, s2_hbm,
        idx1, rows1a, rows1b, out1a, out1b,
        idx2, rows2a, rows2b, out2a, out2b,
        sg0, sg1, so0, so1,
    ):
        wid = _worker_id()
        wbase = wid * PER_W
        sg = (sg0, sg1)
        so = (so0, so1)

        def run_pass(idx_v, rows, outs, idx_hbm, out_hbm, rounds, elems, fanout):
            rows_per = elems * fanout
            pltpu.sync_copy(
                idx_hbm.at[pl.ds(wbase * fanout, PER_W * fanout)], idx_v
            )
            for half in range(2):
                pltpu.async_copy(
                    feats_hbm.at[idx_v.at[pl.ds(half * rows_per, rows_per)]],
                    rows[half], sg[half],
                )

            def body(rr, carry):
                for half in range(2):
                    r = 2 * rr + half
                    buf = rows[half]
                    outb = outs[half]
                    pltpu.make_async_copy(
                        feats_hbm.at[idx_v.at[pl.ds(r * rows_per, rows_per)]],
                        buf, sg[half],
                    ).wait()
                    ebase = wbase + r * elems

                    @pl.when(rr >= 1)
                    def _():
                        pltpu.make_async_copy(
                            outb, out_hbm.at[pl.ds(ebase, elems)], so[half]
                        ).wait()

                    _reduce_rows(buf, outb, elems, fanout)

                    @pl.when(r + 2 < rounds)
                    def _():
                        pltpu.async_copy(
                            feats_hbm.at[
                                idx_v.at[pl.ds((r + 2) * rows_per, rows_per)]
                            ],
                            buf, sg[half],
                        )

                    pltpu.async_copy(
                        outb, out_hbm.at[pl.ds(ebase, elems)], so[half]
                    )
                return carry

            lax.fori_loop(0, rounds // 2, body, 0)
            for half in range(2):
                r = rounds - 2 + half
                pltpu.make_async_copy(
                    outs[half],
                    out_hbm.at[pl.ds(wbase + r * elems, elems)],
                    so[half],
                ).wait()

        run_pass(idx1, (rows1a, rows1b), (out1a, out1b),
                 n1_hbm, s1_hbm, R1, E1, F1)
        run_pass(idx2, (rows2a, rows2b), (out2a, out2b),
                 n2_hbm, s2_hbm, R2, E2, F2)

    return sc_kernel(feats, n1_flat, n2_flat)


def _tc_dense(ht, s1, s2, W1a, W1b, b1, W2a, W2b, b2):
    BLK = 1024
    grid = (B // BLK,)

    def body(ht_r, s1_r, s2_r, w1a_r, w1b_r, b1_r, w2a_r, w2b_r, b2_r, out_r):
        h = ht_r[...]
        a1 = s1_r[...] / 10.0
        x1 = (
            jnp.dot(h, w1a_r[...], preferred_element_type=jnp.float32)
            + jnp.dot(a1, w1b_r[...], preferred_element_type=jnp.float32)
            + b1_r[...]
        )
        h1 = jnp.maximum(x1, 0.0)
        a2 = s2_r[...] / 5.0
        x2 = (
            jnp.dot(h1, w2a_r[...], preferred_element_type=jnp.float32)
            + jnp.dot(a2, w2b_r[...], preferred_element_type=jnp.float32)
            + b2_r[...]
        )
        out_r[...] = jnp.maximum(x2, 0.0)

    row_spec = pl.BlockSpec((BLK, D), lambda i: (i, 0))
    full = lambda shape: pl.BlockSpec(shape, lambda i: tuple(0 for _ in shape))
    return pl.pallas_call(
        body,
        grid=grid,
        in_specs=[
            row_spec, row_spec, row_spec,
            full((D, 64)), full((D, 64)), full((1, 64)),
            full((64, 64)), full((D, 64)), full((1, 64)),
        ],
        out_specs=pl.BlockSpec((BLK, 64), lambda i: (i, 0)),
        out_shape=jax.ShapeDtypeStruct((B, 64), jnp.float32),
    )(ht, s1, s2, W1a, W1b, b1, W2a, W2b, b2)


def kernel(target_node_ids, all_node_features, neighbor_ids_l1, neighbor_ids_l2,
           W1, b1, W2, b2):
    tids = target_node_ids.astype(jnp.int32)
    n1_flat = neighbor_ids_l1.astype(jnp.int32).reshape(-1)
    n2_flat = neighbor_ids_l2.astype(jnp.int32).reshape(-1)
    ht = _sc_gather_targets(tids, all_node_features)
    s1, s2 = _sc_gather_neighbors(all_node_features, n1_flat, n2_flat)
    return _tc_dense(
        ht, s1, s2,
        W1[:D], W1[D:], b1.reshape(1, -1),
        W2[:64], W2[64:], b2.reshape(1, -1),
    )


# speedup vs baseline: 1.0279x; 1.0279x over previous
import functools

import jax
import jax.numpy as jnp
from jax import lax
from jax.experimental import pallas as pl
from jax.experimental.pallas import tpu as pltpu
from jax.experimental.pallas import tpu_sc as plsc

N_NODES = 100000
D = 128
B = 16384
F1 = 10
F2 = 5
NV = D // 16

PER_W = B // 32
TCH = 128
RT = PER_W // TCH
E1 = 8
R1 = PER_W // E1
E2 = 16
R2 = PER_W // E2


def _reduce_rows(rows_v, out_v, n_elems, fanout):
    def elem_body(e, carry):
        base = e * fanout
        accs = [rows_v[base, pl.ds(v * 16, 16)] for v in range(NV)]
        for j in range(1, fanout):
            for v in range(NV):
                accs[v] = accs[v] + rows_v[base + j, pl.ds(v * 16, 16)]
        for v in range(NV):
            out_v[e, pl.ds(v * 16, 16)] = accs[v]
        return carry

    lax.fori_loop(0, n_elems, elem_body, 0)


_MESH = plsc.VectorSubcoreMesh(core_axis_name="c", subcore_axis_name="s")
_INFO = plsc.get_sparse_core_info()


def _worker_id():
    return lax.axis_index("s") * _INFO.num_cores + lax.axis_index("c")


def _sc_gather_targets(tids, feats):
    @functools.partial(
        pl.kernel,
        out_type=[jax.ShapeDtypeStruct((B, D), jnp.float32)],
        mesh=_MESH,
        scratch_types=[
            pltpu.VMEM((PER_W,), jnp.int32),
            pltpu.VMEM((TCH, D), jnp.float32),
            pltpu.VMEM((TCH, D), jnp.float32),
            pltpu.SemaphoreType.DMA,
            pltpu.SemaphoreType.DMA,
            pltpu.SemaphoreType.DMA,
            pltpu.SemaphoreType.DMA,
        ],
    )
    def sc_kernel(tids_hbm, feats_hbm, ht_hbm, idxT, rowsT0, rowsT1,
                  sg0, sg1, so0, so1):
        wid = _worker_id()
        wbase = wid * PER_W
        sg = (sg0, sg1)
        so = (so0, so1)
        rowsT = (rowsT0, rowsT1)
        pltpu.sync_copy(tids_hbm.at[pl.ds(wbase, PER_W)], idxT)
        for half in range(2):
            pltpu.async_copy(
                feats_hbm.at[idxT.at[pl.ds(half * TCH, TCH)]],
                rowsT[half], sg[half],
            )

        def t_body(rr, carry):
            for half in range(2):
                r = 2 * rr + half
                buf = rowsT[half]
                pltpu.make_async_copy(
                    feats_hbm.at[idxT.at[pl.ds(r * TCH, TCH)]], buf, sg[half]
                ).wait()
                base = wbase + r * TCH
                pltpu.async_copy(buf, ht_hbm.at[pl.ds(base, TCH)], so[half])

                @pl.when(r + 2 < RT)
                def _():
                    pltpu.make_async_copy(
                        buf, ht_hbm.at[pl.ds(base, TCH)], so[half]
                    ).wait()
                    pltpu.async_copy(
                        feats_hbm.at[idxT.at[pl.ds((r + 2) * TCH, TCH)]],
                        buf, sg[half],
                    )

            return carry

        lax.fori_loop(0, RT // 2, t_body, 0)
        for half in range(2):
            r = RT - 2 + half
            pltpu.make_async_copy(
                rowsT[half], ht_hbm.at[pl.ds(wbase + r * TCH, TCH)], so[half]
            ).wait()

    return sc_kernel(tids, feats)[0]


def _sc_gather_neighbors(feats, nids_flat, rounds, elems, fanout):
    rows_per = elems * fanout

    @functools.partial(
        pl.kernel,
        out_type=[jax.ShapeDtypeStruct((B, D), jnp.float32)],
        mesh=_MESH,
        scratch_types=[
            pltpu.VMEM((PER_W * fanout,), jnp.int32),
            pltpu.VMEM((rows_per, D), jnp.float32),
            pltpu.VMEM((rows_per, D), jnp.float32),
            pltpu.VMEM((elems, D), jnp.float32),
            pltpu.VMEM((elems, D), jnp.float32),
            pltpu.SemaphoreType.DMA,
            pltpu.SemaphoreType.DMA,
            pltpu.SemaphoreType.DMA,
            pltpu.SemaphoreType.DMA,
        ],
    )
    def sc_kernel(
        feats_hbm, nid_hbm, out_hbm,
        idx_v, rows0, rows1, outb0, outb1,
        sg0, sg1, so0, so1,
    ):
        wid = _worker_id()
        wbase = wid * PER_W
        sg = (sg0, sg1)
        so = (so0, so1)
        rows = (rows0, rows1)
        outs = (outb0, outb1)

        pltpu.sync_copy(
            nid_hbm.at[pl.ds(wbase * fanout, PER_W * fanout)], idx_v
        )
        for half in range(2):
            pltpu.async_copy(
                feats_hbm.at[idx_v.at[pl.ds(half * rows_per, rows_per)]],
                rows[half], sg[half],
            )

        def body(rr, carry):
            for half in range(2):
                r = 2 * rr + half
                buf = rows[half]
                outb = outs[half]
                pltpu.make_async_copy(
                    feats_hbm.at[idx_v.at[pl.ds(r * rows_per, rows_per)]],
                    buf, sg[half],
                ).wait()
                ebase = wbase + r * elems

                @pl.when(rr >= 1)
                def _():
                    pltpu.make_async_copy(
                        outb, out_hbm.at[pl.ds(ebase, elems)], so[half]
                    ).wait()

                _reduce_rows(buf, outb, elems, fanout)

                @pl.when(r + 2 < rounds)
                def _():
                    pltpu.async_copy(
                        feats_hbm.at[
                            idx_v.at[pl.ds((r + 2) * rows_per, rows_per)]
                        ],
                        buf, sg[half],
                    )

                pltpu.async_copy(
                    outb, out_hbm.at[pl.ds(ebase, elems)], so[half]
                )
            return carry

        lax.fori_loop(0, rounds // 2, body, 0)
        for half in range(2):
            r = rounds - 2 + half
            pltpu.make_async_copy(
                outs[half],
                out_hbm.at[pl.ds(wbase + r * elems, elems)],
                so[half],
            ).wait()

    return sc_kernel(feats, nids_flat)[0]


def _tc_dense_layer(x, s, Wa, Wb, b, inv_n):
    BLK = 2048
    grid = (B // BLK,)
    dx = x.shape[1]

    def body(x_r, s_r, wa_r, wb_r, b_r, out_r):
        acc = (
            jnp.dot(x_r[...], wa_r[...], preferred_element_type=jnp.float32)
            + jnp.dot(s_r[...] * inv_n, wb_r[...],
                      preferred_element_type=jnp.float32)
            + b_r[...]
        )
        out_r[...] = jnp.maximum(acc, 0.0)

    full = lambda shape: pl.BlockSpec(shape, lambda i: tuple(0 for _ in shape))
    return pl.pallas_call(
        body,
        grid=grid,
        in_specs=[
            pl.BlockSpec((BLK, dx), lambda i: (i, 0)),
            pl.BlockSpec((BLK, D), lambda i: (i, 0)),
            full((dx, 64)), full((D, 64)), full((1, 64)),
        ],
        out_specs=pl.BlockSpec((BLK, 64), lambda i: (i, 0)),
        out_shape=jax.ShapeDtypeStruct((B, 64), jnp.float32),
    )(x, s, Wa, Wb, b)


def kernel(target_node_ids, all_node_features, neighbor_ids_l1, neighbor_ids_l2,
           W1, b1, W2, b2):
    tids = target_node_ids.astype(jnp.int32)
    n1_flat = neighbor_ids_l1.astype(jnp.int32).reshape(-1)
    n2_flat = neighbor_ids_l2.astype(jnp.int32).reshape(-1)
    ht = _sc_gather_targets(tids, all_node_features)
    s1 = _sc_gather_neighbors(all_node_features, n1_flat, R1, E1, F1)
    s2 = _sc_gather_neighbors(all_node_features, n2_flat, R2, E2, F2)
    h1 = _tc_dense_layer(ht, s1, W1[:D], W1[D:], b1.reshape(1, -1), 1.0 / F1)
    return _tc_dense_layer(h1, s2, W2[:64], W2[64:], b2.reshape(1, -1), 1.0 / F2)


# speedup vs baseline: 1.3643x; 1.3273x over previous
import functools

import jax
import jax.numpy as jnp
from jax import lax
from jax.experimental import pallas as pl
from jax.experimental.pallas import tpu as pltpu
from jax.experimental.pallas import tpu_sc as plsc

N_NODES = 100000
D = 128
B = 16384
F1 = 10
F2 = 5
NV = D // 16

PER_W = B // 32
TCH = 64
NT = PER_W // TCH
E1 = 32
E2 = 64

_MESH = plsc.VectorSubcoreMesh(core_axis_name="c", subcore_axis_name="s")
_INFO = plsc.get_sparse_core_info()


def _worker_id():
    return lax.axis_index("s") * _INFO.num_cores + lax.axis_index("c")


def _reduce_block(bufs, out_v, elems):
    def elem_body(e, carry):
        accs = [bufs[0][e, pl.ds(v * 16, 16)] for v in range(NV)]
        for j in range(1, len(bufs)):
            for v in range(NV):
                accs[v] = accs[v] + bufs[j][e, pl.ds(v * 16, 16)]
        for v in range(NV):
            out_v[e, pl.ds(v * 16, 16)] = accs[v]
        return carry

    lax.fori_loop(0, elems, elem_body, 0)


def _sc_gather_neighbors(feats, nids_T, elems, fanout, tids=None):
    nblk = PER_W // elems
    npair = nblk // 2
    with_t = tids is not None

    out_type = [jax.ShapeDtypeStruct((B, D), jnp.float32)]
    scratch = [pltpu.VMEM((fanout, PER_W), jnp.int32)]
    scratch += [pltpu.VMEM((elems, D), jnp.float32)
                for _ in range(2 * fanout)]
    scratch += [pltpu.VMEM((elems, D), jnp.float32)] * 2
    scratch += [pltpu.SemaphoreType.DMA] * 4
    if with_t:
        out_type.append(jax.ShapeDtypeStruct((B, D), jnp.float32))
        scratch += [
            pltpu.VMEM((PER_W,), jnp.int32),
            pltpu.VMEM((TCH, D), jnp.float32),
            pltpu.VMEM((TCH, D), jnp.float32),
            pltpu.SemaphoreType.DMA,
            pltpu.SemaphoreType.DMA,
            pltpu.SemaphoreType.DMA,
            pltpu.SemaphoreType.DMA,
        ]

    @functools.partial(
        pl.kernel, out_type=out_type, mesh=_MESH, scratch_types=scratch
    )
    def sc_kernel(feats_hbm, nid_hbm, *args):
        args = list(args)
        tids_hbm = args.pop(0) if with_t else None
        out_hbm = args.pop(0)
        ht_hbm = args.pop(0) if with_t else None
        idx_v = args.pop(0)
        setA = [args.pop(0) for _ in range(fanout)]
        setB = [args.pop(0) for _ in range(fanout)]
        outA, outB = args.pop(0), args.pop(0)
        semA, semB, soA, soB = (args.pop(0) for _ in range(4))
        if with_t:
            (idxT, rT0, rT1, tg0, tg1, to0, to1) = args

        wid = _worker_id()
        wbase = wid * PER_W

        def fire_set(bufs, sem, blk):
            for j in range(fanout):
                pltpu.async_copy(
                    feats_hbm.at[idx_v.at[j, pl.ds(blk * elems, elems)]],
                    bufs[j], sem,
                )

        def wait_set(bufs, sem, blk):
            for j in range(fanout):
                pltpu.make_async_copy(
                    feats_hbm.at[idx_v.at[j, pl.ds(blk * elems, elems)]],
                    bufs[j], sem,
                ).wait()

        def fire_out(outb, sem, blk):
            pltpu.async_copy(
                outb, out_hbm.at[pl.ds(wbase + blk * elems, elems)], sem
            )

        def wait_out(outb, sem, blk):
            pltpu.make_async_copy(
                outb, out_hbm.at[pl.ds(wbase + blk * elems, elems)], sem
            ).wait()

        if with_t:
            rT = (rT0, rT1)
            tg = (tg0, tg1)
            to = (to0, to1)

            def t_gather(c, half):
                pltpu.async_copy(
                    feats_hbm.at[idxT.at[pl.ds(c * TCH, TCH)]],
                    rT[half], tg[half],
                )

            def t_wait_g(c, half):
                pltpu.make_async_copy(
                    feats_hbm.at[idxT.at[pl.ds(c * TCH, TCH)]],
                    rT[half], tg[half],
                ).wait()

            def t_out(c, half):
                pltpu.async_copy(
                    rT[half], ht_hbm.at[pl.ds(wbase + c * TCH, TCH)], to[half]
                )

            def t_wait_o(c, half):
                pltpu.make_async_copy(
                    rT[half], ht_hbm.at[pl.ds(wbase + c * TCH, TCH)], to[half]
                ).wait()

        pltpu.sync_copy(
            nid_hbm.at[pl.ds(0, fanout), pl.ds(wbase, PER_W)], idx_v
        )
        if with_t:
            pltpu.sync_copy(tids_hbm.at[pl.ds(wbase, PER_W)], idxT)
            t_gather(0, 0)
            t_gather(1, 1)
        fire_set(setA, semA, 0)
        fire_set(setB, semB, 1)

        def body(t, carry):
            if with_t:
                for k in range(1, NT):
                    @pl.when(t == k)
                    def _(k=k):
                        if k % 2 == 1:
                            t_wait_g(k - 1, 0); t_out(k - 1, 0)
                            t_wait_g(k, 1); t_out(k, 1)
                        else:
                            t_wait_o(k - 2, 0); t_gather(k, 0)
                            t_wait_o(k - 1, 1); t_gather(k + 1, 1)

            for parity, (bufs, sem, outb, so) in enumerate(
                ((setA, semA, outA, soA), (setB, semB, outB, soB))
            ):
                blk = 2 * t + parity
                wait_set(bufs, sem, blk)

                @pl.when(t >= 1)
                def _():
                    wait_out(outb, so, blk)

                _reduce_block(bufs, outb, elems)

                @pl.when(t + 1 < npair)
                def _():
                    fire_set(bufs, sem, blk + 2)

                fire_out(outb, so, blk)
            return carry

        lax.fori_loop(0, npair, body, 0)
        wait_out(outA, soA, 2 * npair - 2)
        wait_out(outB, soB, 2 * npair - 1)
        if with_t:
            t_wait_o(NT - 2, 0)
            t_wait_o(NT - 1, 1)

    if with_t:
        return sc_kernel(feats, nids_T, tids)
    return sc_kernel(feats, nids_T)[0]


def _tc_dense_layer(x, s, Wa, Wb, b, inv_n):
    BLK = 2048
    grid = (B // BLK,)
    dx = x.shape[1]

    def body(x_r, s_r, wa_r, wb_r, b_r, out_r):
        acc = (
            jnp.dot(x_r[...], wa_r[...], preferred_element_type=jnp.float32)
            + jnp.dot(s_r[...] * inv_n, wb_r[...],
                      preferred_element_type=jnp.float32)
            + b_r[...]
        )
        out_r[...] = jnp.maximum(acc, 0.0)

    full = lambda shape: pl.BlockSpec(shape, lambda i: tuple(0 for _ in shape))
    return pl.pallas_call(
        body,
        grid=grid,
        in_specs=[
            pl.BlockSpec((BLK, dx), lambda i: (i, 0)),
            pl.BlockSpec((BLK, D), lambda i: (i, 0)),
            full((dx, 64)), full((D, 64)), full((1, 64)),
        ],
        out_specs=pl.BlockSpec((BLK, 64), lambda i: (i, 0)),
        out_shape=jax.ShapeDtypeStruct((B, 64), jnp.float32),
    )(x, s, Wa, Wb, b)


def kernel(target_node_ids, all_node_features, neighbor_ids_l1, neighbor_ids_l2,
           W1, b1, W2, b2):
    tids = target_node_ids.astype(jnp.int32)
    n1_T = neighbor_ids_l1.astype(jnp.int32).T
    n2_T = neighbor_ids_l2.astype(jnp.int32).T
    s1, ht = _sc_gather_neighbors(all_node_features, n1_T, E1, F1, tids=tids)
    s2 = _sc_gather_neighbors(all_node_features, n2_T, E2, F2)
    h1 = _tc_dense_layer(ht, s1, W1[:D], W1[D:], b1.reshape(1, -1), 1.0 / F1)
    return _tc_dense_layer(h1, s2, W2[:64], W2[64:], b2.reshape(1, -1), 1.0 / F2)
